# baseline (speedup 1.0000x reference)
import functools

import jax
import jax.numpy as jnp
from jax import lax
from jax.experimental import pallas as pl
from jax.experimental.pallas import tpu as pltpu
from jax.experimental.pallas import tpu_sc as plsc

DIM = 16
BATCH = 16384
PACK = 8
LINE = PACK * DIM
NUM_CORES = 2
NUM_SUBCORES = 16
NW = NUM_CORES * NUM_SUBCORES
CHUNK = 128
ROWS_PER_W = BATCH // (NW * CHUNK)
IDX_ROWS = BATCH // CHUNK
NJOBS = 2 * ROWS_PER_W
NGRP = CHUNK // 16


def _sc_gather_body(ubase_hbm, ibase_hbm, uoff_hbm, ioff_hbm,
                    uemb_hbm, iemb_hbm, u_out, i_out,
                    uidx_v, iidx_v, uoff_v, ioff_v, buf_v, ustage, istage,
                    sem0, sem1):
    wid = lax.axis_index("s") * NUM_CORES + lax.axis_index("c")
    base = wid * ROWS_PER_W
    pltpu.sync_copy(ubase_hbm.at[pl.ds(base, ROWS_PER_W)], uidx_v)
    pltpu.sync_copy(ibase_hbm.at[pl.ds(base, ROWS_PER_W)], iidx_v)
    pltpu.sync_copy(uoff_hbm.at[pl.ds(base, ROWS_PER_W)], uoff_v)
    pltpu.sync_copy(ioff_hbm.at[pl.ds(base, ROWS_PER_W)], ioff_v)

    jobs = [(uemb_hbm, uidx_v, uoff_v, ustage, j) for j in range(ROWS_PER_W)] \
         + [(iemb_hbm, iidx_v, ioff_v, istage, j) for j in range(ROWS_PER_W)]
    sems = (sem0, sem1)

    def start(k):
        emb, idx, _, _, j = jobs[k]
        return pltpu.async_copy(emb.at[idx.at[j]], buf_v.at[k % 2], sems[k % 2])

    pend = [start(0), start(1)]
    for k in range(NJOBS):
        _, _, off, stage, j = jobs[k]
        pend[k % 2].wait()
        buf2 = buf_v.at[k % 2]
        out2 = stage.at[j]

        def grp(g, _, off=off, buf2=buf2, out2=out2, j=j):
            s = g * 16 + lax.iota(jnp.int32, 16)
            offv = off[j, pl.ds(g * 16, 16)]
            for c in range(DIM):
                val = plsc.load_gather(buf2, [s, offv + c])
                flat = s * DIM + c
                plsc.store_scatter(out2, [flat >> 7, flat & 127], val)
            return 0

        lax.fori_loop(0, NGRP, grp, 0)
        if k + 2 < NJOBS:
            pend[k % 2] = start(k + 2)

    pltpu.sync_copy(ustage, u_out.at[pl.ds(base, ROWS_PER_W)])
    pltpu.sync_copy(istage, i_out.at[pl.ds(base, ROWS_PER_W)])


_sc_gather = functools.partial(
    pl.kernel,
    out_type=(
        jax.ShapeDtypeStruct((IDX_ROWS, CHUNK * DIM // 128, 128), jnp.float32),
        jax.ShapeDtypeStruct((IDX_ROWS, CHUNK * DIM // 128, 128), jnp.float32),
    ),
    mesh=plsc.VectorSubcoreMesh(core_axis_name="c", subcore_axis_name="s"),
    compiler_params=pltpu.CompilerParams(use_tc_tiling_on_sc=True,
                                         needs_layout_passes=False),
    scratch_types=[
        pltpu.VMEM((ROWS_PER_W, CHUNK), jnp.int32),
        pltpu.VMEM((ROWS_PER_W, CHUNK), jnp.int32),
        pltpu.VMEM((ROWS_PER_W, CHUNK), jnp.int32),
        pltpu.VMEM((ROWS_PER_W, CHUNK), jnp.int32),
        pltpu.VMEM((2, CHUNK, LINE), jnp.float32),
        pltpu.VMEM((ROWS_PER_W, CHUNK * DIM // 128, 128), jnp.float32),
        pltpu.VMEM((ROWS_PER_W, CHUNK * DIM // 128, 128), jnp.float32),
        pltpu.SemaphoreType.DMA,
        pltpu.SemaphoreType.DMA,
    ],
)(_sc_gather_body)


def _mlp_body(u_ref, i_ref, w1u_ref, w1i_ref, b1_ref, w2_ref, b2_ref, out_ref):
    h = jnp.dot(u_ref[...], w1u_ref[...], preferred_element_type=jnp.float32)
    h = h + jnp.dot(i_ref[...], w1i_ref[...], preferred_element_type=jnp.float32)
    h = jnp.maximum(h + b1_ref[...], 0.0)
    out_ref[...] = jnp.dot(h, w2_ref[...], preferred_element_type=jnp.float32) + b2_ref[0, 0]


def _mlp(u, i, w1u, w1i, b1, w2, b2):
    return pl.pallas_call(
        _mlp_body,
        out_shape=jax.ShapeDtypeStruct((BATCH, 1), jnp.float32),
    )(u, i, w1u, w1i, b1, w2, b2)


def kernel(user, item, user_emb, item_emb, W1, b1, W2, b2):
    user = user.astype(jnp.int32)
    item = item.astype(jnp.int32)
    ubase = (user >> 3).reshape(IDX_ROWS, CHUNK)
    ibase = (item >> 3).reshape(IDX_ROWS, CHUNK)
    uoff = ((user & 7) * DIM).reshape(IDX_ROWS, CHUNK)
    ioff = ((item & 7) * DIM).reshape(IDX_ROWS, CHUNK)
    uemb = user_emb.reshape(-1, LINE)
    iemb = item_emb.reshape(-1, LINE)
    u, i = _sc_gather(ubase, ibase, uoff, ioff, uemb, iemb)
    u = u.reshape(BATCH, DIM)
    i = i.reshape(BATCH, DIM)
    w1u = W1[:, :DIM].T
    w1i = W1[:, DIM:].T
    out = _mlp(u, i, w1u, w1i, b1.reshape(1, 32), W2.T, b2.reshape(1, 1))
    return out.reshape(BATCH)


# speedup vs baseline: 1.4563x; 1.4563x over previous
import functools

import jax
import jax.numpy as jnp
from jax import lax
from jax.experimental import pallas as pl
from jax.experimental.pallas import tpu as pltpu
from jax.experimental.pallas import tpu_sc as plsc

DIM = 16
BATCH = 16384
PACK = 8
LINE = PACK * DIM
NUM_CORES = 2
NUM_SUBCORES = 16
NW = NUM_CORES * NUM_SUBCORES
CHUNK = 128
ROWS_PER_W = BATCH // (NW * CHUNK)
IDX_ROWS = BATCH // CHUNK
NJOBS = 2 * ROWS_PER_W
NGRP = CHUNK // 16


def _sc_gather_body(ubase_hbm, ibase_hbm, uoff_hbm, ioff_hbm,
                    uemb_hbm, iemb_hbm, u_out, i_out,
                    uidx_v, iidx_v, uoff_v, ioff_v, buf_v, ustage, istage,
                    sem0, sem1):
    wid = lax.axis_index("s") * NUM_CORES + lax.axis_index("c")
    base = wid * ROWS_PER_W
    pltpu.sync_copy(ubase_hbm.at[pl.ds(base, ROWS_PER_W)], uidx_v)
    pltpu.sync_copy(ibase_hbm.at[pl.ds(base, ROWS_PER_W)], iidx_v)
    pltpu.sync_copy(uoff_hbm.at[pl.ds(base, ROWS_PER_W)], uoff_v)
    pltpu.sync_copy(ioff_hbm.at[pl.ds(base, ROWS_PER_W)], ioff_v)

    jobs = [(uemb_hbm, uidx_v, uoff_v, ustage, j) for j in range(ROWS_PER_W)] \
         + [(iemb_hbm, iidx_v, ioff_v, istage, j) for j in range(ROWS_PER_W)]
    sems = (sem0, sem1)

    def start(k):
        emb, idx, _, _, j = jobs[k]
        return pltpu.async_copy(emb.at[idx.at[j]], buf_v.at[k % 2], sems[k % 2])

    pend = [start(0), start(1)]
    for k in range(NJOBS):
        _, _, off, stage, j = jobs[k]
        pend[k % 2].wait()
        buf2 = buf_v.at[k % 2]
        out2 = stage.at[j]

        def grp(g, _, off=off, buf2=buf2, out2=out2, j=j):
            s = g * 16 + lax.iota(jnp.int32, 16)
            offv = off[j, pl.ds(g * 16, 16)]
            for c in range(DIM):
                val = plsc.load_gather(buf2, [s, offv + c])
                flat = s * DIM + c
                plsc.store_scatter(out2, [flat >> 7, flat & 127], val)
            return 0

        lax.fori_loop(0, NGRP, grp, 0)
        if k + 2 < NJOBS:
            pend[k % 2] = start(k + 2)

    pltpu.sync_copy(ustage, u_out.at[pl.ds(base, ROWS_PER_W)])
    pltpu.sync_copy(istage, i_out.at[pl.ds(base, ROWS_PER_W)])


_sc_gather = functools.partial(
    pl.kernel,
    out_type=(
        jax.ShapeDtypeStruct((IDX_ROWS, CHUNK * DIM // 128, 128), jnp.float32),
        jax.ShapeDtypeStruct((IDX_ROWS, CHUNK * DIM // 128, 128), jnp.float32),
    ),
    mesh=plsc.VectorSubcoreMesh(core_axis_name="c", subcore_axis_name="s"),
    compiler_params=pltpu.CompilerParams(use_tc_tiling_on_sc=True,
                                         needs_layout_passes=False),
    scratch_types=[
        pltpu.VMEM((ROWS_PER_W, CHUNK), jnp.int32),
        pltpu.VMEM((ROWS_PER_W, CHUNK), jnp.int32),
        pltpu.VMEM((ROWS_PER_W, CHUNK), jnp.int32),
        pltpu.VMEM((ROWS_PER_W, CHUNK), jnp.int32),
        pltpu.VMEM((2, CHUNK, LINE), jnp.float32),
        pltpu.VMEM((ROWS_PER_W, CHUNK * DIM // 128, 128), jnp.float32),
        pltpu.VMEM((ROWS_PER_W, CHUNK * DIM // 128, 128), jnp.float32),
        pltpu.SemaphoreType.DMA,
        pltpu.SemaphoreType.DMA,
    ],
)(_sc_gather_body)


RB = 8192
NLINES = RB // PACK
RGRID = (1000000 + RB - 1) // RB


def _repack_body(ut_ref, it_ref, uo_ref, io_ref):
    eye = (lax.broadcasted_iota(jnp.int32, (DIM, DIM), 0)
           == lax.broadcasted_iota(jnp.int32, (DIM, DIM), 1)).astype(jnp.float32)
    for src, dst in ((ut_ref, uo_ref), (it_ref, io_ref)):
        x = src[...]
        xt = lax.dot_general(x, eye, (((0,), (0,)), ((), ())),
                             preferred_element_type=jnp.float32)
        x3 = xt.reshape(NLINES, PACK, DIM)
        for s in range(PACK):
            dst[:, s * DIM:(s + 1) * DIM] = x3[:, s, :]


def _repack(uembT, iembT):
    return pl.pallas_call(
        _repack_body,
        grid=(RGRID,),
        in_specs=[
            pl.BlockSpec((DIM, RB), lambda n: (0, n)),
            pl.BlockSpec((DIM, RB), lambda n: (0, n)),
        ],
        out_specs=[
            pl.BlockSpec((NLINES, LINE), lambda n: (n, 0)),
            pl.BlockSpec((NLINES, LINE), lambda n: (n, 0)),
        ],
        out_shape=[
            jax.ShapeDtypeStruct((1000000 // PACK, LINE), jnp.float32),
            jax.ShapeDtypeStruct((1000000 // PACK, LINE), jnp.float32),
        ],
    )(uembT, iembT)


def _mlp_body(u_ref, i_ref, w1u_ref, w1i_ref, b1_ref, w2_ref, b2_ref, out_ref):
    h = jnp.dot(u_ref[...], w1u_ref[...], preferred_element_type=jnp.float32)
    h = h + jnp.dot(i_ref[...], w1i_ref[...], preferred_element_type=jnp.float32)
    h = jnp.maximum(h + b1_ref[...], 0.0)
    out_ref[...] = jnp.dot(h, w2_ref[...], preferred_element_type=jnp.float32) + b2_ref[0, 0]


def _mlp(u, i, w1u, w1i, b1, w2, b2):
    return pl.pallas_call(
        _mlp_body,
        out_shape=jax.ShapeDtypeStruct((BATCH, 1), jnp.float32),
    )(u, i, w1u, w1i, b1, w2, b2)


def kernel(user, item, user_emb, item_emb, W1, b1, W2, b2):
    user = user.astype(jnp.int32)
    item = item.astype(jnp.int32)
    ubase = (user >> 3).reshape(IDX_ROWS, CHUNK)
    ibase = (item >> 3).reshape(IDX_ROWS, CHUNK)
    uoff = ((user & 7) * DIM).reshape(IDX_ROWS, CHUNK)
    ioff = ((item & 7) * DIM).reshape(IDX_ROWS, CHUNK)
    uemb, iemb = _repack(user_emb.T, item_emb.T)
    u, i = _sc_gather(ubase, ibase, uoff, ioff, uemb, iemb)
    u = u.reshape(BATCH, DIM)
    i = i.reshape(BATCH, DIM)
    w1u = W1[:, :DIM].T
    w1i = W1[:, DIM:].T
    out = _mlp(u, i, w1u, w1i, b1.reshape(1, 32), W2.T, b2.reshape(1, 1))
    return out.reshape(BATCH)


# speedup vs baseline: 1.5237x; 1.0463x over previous
import functools

import jax
import jax.numpy as jnp
from jax import lax
from jax.experimental import pallas as pl
from jax.experimental.pallas import tpu as pltpu
from jax.experimental.pallas import tpu_sc as plsc

DIM = 16
BATCH = 16384
PACK = 8
LINE = PACK * DIM
NUM_CORES = 2
NUM_SUBCORES = 16
NW = NUM_CORES * NUM_SUBCORES
CHUNK = 128
ROWS_PER_W = BATCH // (NW * CHUNK)
IDX_ROWS = BATCH // CHUNK
NJOBS = 2 * ROWS_PER_W
NGRP = CHUNK // 16


def _sc_gather_body(ubase_hbm, ibase_hbm, uoff_hbm, ioff_hbm,
                    uemb_hbm, iemb_hbm, u_out, i_out,
                    uidx_v, iidx_v, uoff_v, ioff_v, buf_v, ustage, istage,
                    sem0, sem1):
    wid = lax.axis_index("s") * NUM_CORES + lax.axis_index("c")
    base = wid * ROWS_PER_W
    pltpu.sync_copy(ubase_hbm.at[pl.ds(base, ROWS_PER_W)], uidx_v)
    pltpu.sync_copy(ibase_hbm.at[pl.ds(base, ROWS_PER_W)], iidx_v)
    pltpu.sync_copy(uoff_hbm.at[pl.ds(base, ROWS_PER_W)], uoff_v)
    pltpu.sync_copy(ioff_hbm.at[pl.ds(base, ROWS_PER_W)], ioff_v)

    jobs = [(uemb_hbm, uidx_v, uoff_v, ustage, j) for j in range(ROWS_PER_W)] \
         + [(iemb_hbm, iidx_v, ioff_v, istage, j) for j in range(ROWS_PER_W)]
    sems = (sem0, sem1)

    def start(k):
        emb, idx, _, _, j = jobs[k]
        return pltpu.async_copy(emb.at[idx.at[j]], buf_v.at[k % 2], sems[k % 2])

    pend = [start(0), start(1)]
    for k in range(NJOBS):
        _, _, off, stage, j = jobs[k]
        pend[k % 2].wait()
        buf2 = buf_v.at[k % 2]
        out2 = stage.at[pl.ds(j * (CHUNK * DIM // 128), CHUNK * DIM // 128)]

        def grp(g, _, off=off, buf2=buf2, out2=out2, j=j):
            s = g * 16 + lax.iota(jnp.int32, 16)
            offv = off[j, pl.ds(g * 16, 16)]
            for c in range(DIM):
                val = plsc.load_gather(buf2, [s, offv + c])
                flat = s * DIM + c
                plsc.store_scatter(out2, [flat >> 7, flat & 127], val)
            return 0

        lax.fori_loop(0, NGRP, grp, 0)
        if k + 2 < NJOBS:
            pend[k % 2] = start(k + 2)

    nlw = ROWS_PER_W * CHUNK * DIM // 128
    pltpu.sync_copy(ustage, u_out.at[pl.ds(wid * nlw, nlw)])
    pltpu.sync_copy(istage, i_out.at[pl.ds(wid * nlw, nlw)])


_sc_gather = functools.partial(
    pl.kernel,
    out_type=(
        jax.ShapeDtypeStruct((BATCH * DIM // 128, 128), jnp.float32),
        jax.ShapeDtypeStruct((BATCH * DIM // 128, 128), jnp.float32),
    ),
    mesh=plsc.VectorSubcoreMesh(core_axis_name="c", subcore_axis_name="s"),
    compiler_params=pltpu.CompilerParams(use_tc_tiling_on_sc=True,
                                         needs_layout_passes=False),
    scratch_types=[
        pltpu.VMEM((ROWS_PER_W, CHUNK), jnp.int32),
        pltpu.VMEM((ROWS_PER_W, CHUNK), jnp.int32),
        pltpu.VMEM((ROWS_PER_W, CHUNK), jnp.int32),
        pltpu.VMEM((ROWS_PER_W, CHUNK), jnp.int32),
        pltpu.VMEM((2, CHUNK, LINE), jnp.float32),
        pltpu.VMEM((ROWS_PER_W * CHUNK * DIM // 128, 128), jnp.float32),
        pltpu.VMEM((ROWS_PER_W * CHUNK * DIM // 128, 128), jnp.float32),
        pltpu.SemaphoreType.DMA,
        pltpu.SemaphoreType.DMA,
    ],
)(_sc_gather_body)


RB = 8192
NLINES = RB // PACK
RGRID = (1000000 + RB - 1) // RB


def _repack_body(ut_ref, it_ref, uo_ref, io_ref):
    for src, dst in ((ut_ref, uo_ref), (it_ref, io_ref)):
        x = src[...]
        xt = x.T
        x3 = xt.reshape(NLINES, PACK, DIM)
        for s in range(PACK):
            dst[:, s * DIM:(s + 1) * DIM] = x3[:, s, :]


def _repack(uembT, iembT):
    return pl.pallas_call(
        _repack_body,
        grid=(RGRID,),
        in_specs=[
            pl.BlockSpec((DIM, RB), lambda n: (0, n)),
            pl.BlockSpec((DIM, RB), lambda n: (0, n)),
        ],
        out_specs=[
            pl.BlockSpec((NLINES, LINE), lambda n: (n, 0)),
            pl.BlockSpec((NLINES, LINE), lambda n: (n, 0)),
        ],
        out_shape=[
            jax.ShapeDtypeStruct((1000000 // PACK, LINE), jnp.float32),
            jax.ShapeDtypeStruct((1000000 // PACK, LINE), jnp.float32),
        ],
    )(uembT, iembT)


def _mlp_body(u_ref, i_ref, wu_ref, wi_ref, b1_ref, w2_ref, b2_ref, out_ref):
    h = jnp.dot(u_ref[...], wu_ref[...], preferred_element_type=jnp.float32)
    h = h + jnp.dot(i_ref[...], wi_ref[...], preferred_element_type=jnp.float32)
    h = jnp.maximum(h + b1_ref[...], 0.0)
    out_ref[...] = jnp.dot(h, w2_ref[...], preferred_element_type=jnp.float32) + b2_ref[0, 0]


def _mlp(u, i, wu, wi, b1big, w2big, b2):
    return pl.pallas_call(
        _mlp_body,
        out_shape=jax.ShapeDtypeStruct((BATCH * DIM // 128, PACK), jnp.float32),
    )(u, i, wu, wi, b1big, w2big, b2)


def kernel(user, item, user_emb, item_emb, W1, b1, W2, b2):
    user = user.astype(jnp.int32)
    item = item.astype(jnp.int32)
    ubase = (user >> 3).reshape(IDX_ROWS, CHUNK)
    ibase = (item >> 3).reshape(IDX_ROWS, CHUNK)
    uoff = ((user & 7) * DIM).reshape(IDX_ROWS, CHUNK)
    ioff = ((item & 7) * DIM).reshape(IDX_ROWS, CHUNK)
    uemb, iemb = _repack(user_emb.T, item_emb.T)
    u, i = _sc_gather(ubase, ibase, uoff, ioff, uemb, iemb)
    eye8 = jnp.eye(PACK, dtype=jnp.float32)
    wu = jnp.kron(eye8, W1[:, :DIM].T)
    wi = jnp.kron(eye8, W1[:, DIM:].T)
    w2big = jnp.kron(eye8, W2.T)
    b1big = jnp.tile(b1, PACK).reshape(1, PACK * 32)
    out = _mlp(u, i, wu, wi, b1big, w2big, b2.reshape(1, 1))
    return out.reshape(BATCH)
